# Initial kernel scaffold; baseline (speedup 1.0000x reference)
import jax
import jax.numpy as jnp
from jax.experimental import pallas as pl


def kernel(x):
    raise NotImplementedError("write your pallas kernel here")


# speedup vs baseline: 10.3768x; 10.3768x over previous
import functools

import jax
import jax.numpy as jnp
from jax import lax
from jax.experimental import pallas as pl
from jax.experimental.pallas import tpu as pltpu
from jax.experimental.pallas import tpu_sc as plsc

BATCH = 64
N_NODES = 4096
F = 128
LANES = 16
FV = F // LANES

NUM_CORES = 2
NUM_SUBCORES = 16
NUM_WORKERS = NUM_CORES * NUM_SUBCORES
G_PER_W = BATCH // NUM_WORKERS

ROWS = 256
NCHUNK = N_NODES // ROWS
NBUF = 2
UNROLL = 4


def _pool_body(x_hbm, out_hbm, buf, accv, sems):
    wid = lax.axis_index("s") * NUM_CORES + lax.axis_index("c")
    g0 = wid * G_PER_W
    ntot = G_PER_W * NCHUNK

    def src(t):
        g, c = divmod(t, NCHUNK)
        return x_hbm.at[g0 + g, pl.ds(c * ROWS, ROWS), :]

    copies = {0: pltpu.async_copy(src(0), buf.at[0], sems.at[0])}
    acc = None
    for t in range(ntot):
        b = t % NBUF
        if t + 1 < ntot:
            nb = (t + 1) % NBUF
            copies[t + 1] = pltpu.async_copy(src(t + 1), buf.at[nb], sems.at[nb])
        copies[t].wait()
        g, c = divmod(t, NCHUNK)
        if c == 0:
            acc = tuple(jnp.full((LANES,), -jnp.inf, jnp.float32)
                        for _ in range(FV))

        def row_body(r, a, b=b):
            base = r * UNROLL
            new = list(a)
            for u in range(UNROLL):
                for j in range(FV):
                    new[j] = jnp.maximum(
                        new[j], buf[b, base + u, pl.ds(j * LANES, LANES)])
            return tuple(new)

        acc = lax.fori_loop(0, ROWS // UNROLL, row_body, acc)
        if c == NCHUNK - 1:
            for j in range(FV):
                accv[pl.ds(j * LANES, LANES)] = acc[j]
            pltpu.sync_copy(accv, out_hbm.at[g0 + g])


@jax.jit
def _pool(x):
    mesh = plsc.VectorSubcoreMesh(core_axis_name="c", subcore_axis_name="s")
    return pl.kernel(
        _pool_body,
        mesh=mesh,
        out_type=jax.ShapeDtypeStruct((BATCH, F), jnp.float32),
        scratch_types=[
            pltpu.VMEM((NBUF, ROWS, F), jnp.float32),
            pltpu.VMEM((F,), jnp.float32),
            pltpu.SemaphoreType.DMA((NBUF,)),
        ],
    )(x)


def kernel(x):
    return _pool(x)


# speedup vs baseline: 10.8755x; 1.0481x over previous
import functools

import jax
import jax.numpy as jnp
from jax import lax
from jax.experimental import pallas as pl
from jax.experimental.pallas import tpu as pltpu
from jax.experimental.pallas import tpu_sc as plsc

BATCH = 64
N_NODES = 4096
F = 128
LANES = 16
FV = F // LANES

NUM_CORES = 2
NUM_SUBCORES = 16
NUM_WORKERS = NUM_CORES * NUM_SUBCORES
G_PER_W = BATCH // NUM_WORKERS

ROWS = 256
NCHUNK = N_NODES // ROWS
NBUF = 3
UNROLL = 8


def _pool_body(x_hbm, out_hbm, buf, accv, sems):
    wid = lax.axis_index("s") * NUM_CORES + lax.axis_index("c")
    g0 = wid * G_PER_W
    ntot = G_PER_W * NCHUNK

    def src(t):
        g, c = divmod(t, NCHUNK)
        return x_hbm.at[g0 + g, pl.ds(c * ROWS, ROWS), :]

    copies = {t: pltpu.async_copy(src(t), buf.at[t], sems.at[t])
              for t in range(NBUF - 1)}
    acc = None
    for t in range(ntot):
        b = t % NBUF
        nxt = t + NBUF - 1
        if nxt < ntot:
            nb = nxt % NBUF
            copies[nxt] = pltpu.async_copy(src(nxt), buf.at[nb], sems.at[nb])
        copies[t].wait()
        g, c = divmod(t, NCHUNK)
        if c == 0:
            acc = tuple(jnp.full((LANES,), -jnp.inf, jnp.float32)
                        for _ in range(FV))

        def row_body(r, a, b=b):
            base = r * UNROLL
            new = list(a)
            for u in range(UNROLL):
                for j in range(FV):
                    new[j] = jnp.maximum(
                        new[j], buf[b, base + u, pl.ds(j * LANES, LANES)])
            return tuple(new)

        acc = lax.fori_loop(0, ROWS // UNROLL, row_body, acc)
        if c == NCHUNK - 1:
            for j in range(FV):
                accv[pl.ds(j * LANES, LANES)] = acc[j]
            pltpu.sync_copy(accv, out_hbm.at[g0 + g])


@jax.jit
def _pool(x):
    mesh = plsc.VectorSubcoreMesh(core_axis_name="c", subcore_axis_name="s")
    return pl.kernel(
        _pool_body,
        mesh=mesh,
        out_type=jax.ShapeDtypeStruct((BATCH, F), jnp.float32),
        scratch_types=[
            pltpu.VMEM((NBUF, ROWS, F), jnp.float32),
            pltpu.VMEM((F,), jnp.float32),
            pltpu.SemaphoreType.DMA((NBUF,)),
        ],
    )(x)


def kernel(x):
    return _pool(x)
